# Initial kernel scaffold; baseline (speedup 1.0000x reference)
import jax
import jax.numpy as jnp
from jax.experimental import pallas as pl


def kernel(x, ei, pos, feat, W0, b0, W1, b1, Wm1, bm1, Wm2, bm2, Wm3, bm3, Wdir, bdir):
    raise NotImplementedError("write your pallas kernel here")


# speedup vs baseline: 13.0482x; 13.0482x over previous
import jax
import jax.numpy as jnp
from jax.experimental import pallas as pl

N = 512
E = 8192
P = 4096
M = 20
EBLK = 1024
QBLK = 512
NEB = E // EBLK
NQB = (2 * P) // QBLK


def _k1(row_ref, col_ref, feat_ref, w0_ref, b0_ref, w1_ref, b1_ref,
        wm1_ref, bm1_ref, wm2_ref, bm2_ref,
        adjf_ref, adjt_ref, u1_ref, v1_ref, u2_ref, v2_ref, w_ref, h_ref):
    e = pl.program_id(0)
    rows = row_ref[0, 0, :]
    cols = col_ref[0, 0, :]
    ids = jax.lax.broadcasted_iota(jnp.int32, (EBLK, N), 1)
    ohr = (rows[:, None] == ids).astype(jnp.bfloat16)
    ohc = (cols[:, None] == ids).astype(jnp.bfloat16)
    dn = (((0,), (0,)), ((), ()))
    pa = jax.lax.dot_general(ohr, ohc, dn,
                             preferred_element_type=jnp.float32
                             ).astype(jnp.bfloat16)
    pt = jax.lax.dot_general(ohc, ohr, dn,
                             preferred_element_type=jnp.float32
                             ).astype(jnp.bfloat16)

    @pl.when(e == 0)
    def _():
        adjf_ref[...] = pa
        adjt_ref[...] = pt

    @pl.when(e > 0)
    def _():
        adjf_ref[...] += pa
        adjt_ref[...] += pt

    @pl.when(e == NEB - 1)
    def _():
        adjt = adjt_ref[...].astype(jnp.float32)
        deg = jnp.sum(adjt, axis=1) + 1.0
        dinv = 1.0 / jnp.sqrt(deg)
        ir = jax.lax.broadcasted_iota(jnp.int32, (N, N), 0)
        ic = jax.lax.broadcasted_iota(jnp.int32, (N, N), 1)
        eye = (ir == ic).astype(jnp.float32)
        prop = (adjt + eye) * (dinv[:, None] * dinv[None, :])
        f32 = jnp.float32
        h = jnp.dot(prop, jnp.dot(feat_ref[...], w0_ref[...],
                                  preferred_element_type=f32),
                    preferred_element_type=f32) + b0_ref[...]
        h = jnp.dot(prop, jnp.dot(h, w1_ref[...],
                                  preferred_element_type=f32),
                    preferred_element_type=f32) + b1_ref[...]
        wm1 = wm1_ref[...]
        wm2 = wm2_ref[...]
        u1 = jnp.dot(h, wm1[:M, :], preferred_element_type=f32) + bm1_ref[...]
        v1 = jnp.dot(h, wm1[M:, :], preferred_element_type=f32)
        u2 = jnp.dot(h, wm2[:M, :], preferred_element_type=f32)
        v2 = jnp.dot(h, wm2[M:, :], preferred_element_type=f32) + bm2_ref[...]
        u1_ref[...] = u1
        v1_ref[...] = v1
        u2_ref[...] = u2
        v2_ref[...] = v2
        w_ref[...] = v1 * u2
        h_ref[...] = h


def _k2(iq_ref, jq_ref, adjf_ref, adjt_ref, u1_ref, v1_ref, u2_ref, v2_ref,
        w_ref, h_ref, wm3_ref, bm3_ref, aval_ref, xx_ref):
    f32 = jnp.float32
    iqv = iq_ref[0, 0, :]
    jqv = jq_ref[0, 0, :]
    ids = jax.lax.broadcasted_iota(jnp.int32, (QBLK, N), 1)
    ohi = (iqv[:, None] == ids)
    ohj = (jqv[:, None] == ids)
    ohif = ohi.astype(f32)
    ohjf = ohj.astype(f32)
    aqr = jnp.dot(ohi.astype(jnp.bfloat16), adjf_ref[...],
                  preferred_element_type=f32)
    aqc = jnp.dot(ohj.astype(jnp.bfloat16), adjt_ref[...],
                  preferred_element_type=f32)
    z = aqr * aqc
    cnt = jnp.sum(z, axis=1)
    s1 = jnp.dot(z, v1_ref[...], preferred_element_type=f32)
    s2 = jnp.dot(z, u2_ref[...], preferred_element_type=f32)
    sx = jnp.dot(z, w_ref[...], preferred_element_type=f32)
    u1q = jnp.dot(ohif, u1_ref[...], preferred_element_type=f32)
    v2q = jnp.dot(ohjf, v2_ref[...], preferred_element_type=f32)
    hr = jnp.dot(ohif, h_ref[...], preferred_element_type=f32)
    hc = jnp.dot(ohjf, h_ref[...], preferred_element_type=f32)
    adjflag = jnp.sum(aqr * ohjf, axis=1)
    pd = (u1q * v2q) * cnt[:, None] + u1q * s2 + v2q * s1 + sx
    union = ((cnt + adjflag) > 0.0).astype(f32)
    wm3 = wm3_ref[...]
    aval = (jnp.dot(pd, wm3[:M, :], preferred_element_type=f32)
            + adjflag[:, None] * wm3[M, :][None, :] + bm3_ref[...])
    aval_ref[...] = aval * union[:, None]
    xx_ref[...] = hr * hc


def _k3(aval_ref, xx_ref, wdir_ref, bdir_ref, out_ref):
    f32 = jnp.float32
    xf = aval_ref[0:P, :] * aval_ref[P:2 * P, :]
    wdir = wdir_ref[...]
    out_ref[...] = (jnp.dot(xf, wdir[:M, :], preferred_element_type=f32)
                    + jnp.dot(xx_ref[0:P, :], wdir[M:, :],
                              preferred_element_type=f32)
                    + bdir_ref[...])


def kernel(x, ei, pos, feat, W0, b0, W1, b1, Wm1, bm1, Wm2, bm2,
           Wm3, bm3, Wdir, bdir):
    f32 = jnp.float32
    row = ei[0].reshape(NEB, 1, EBLK)
    col = ei[1].reshape(NEB, 1, EBLK)
    iq = jnp.concatenate([pos[:, 0], pos[:, 1]]).reshape(NQB, 1, QBLK)
    jq = jnp.concatenate([pos[:, 1], pos[:, 0]]).reshape(NQB, 1, QBLK)

    full = lambda shp: pl.BlockSpec(shp, lambda *_: tuple(0 for _ in shp))
    ebk = pl.BlockSpec((1, 1, EBLK), lambda e: (e, 0, 0))
    qbk = pl.BlockSpec((1, 1, QBLK), lambda q: (q, 0, 0))

    adjf, adjt, u1, v1, u2, v2, w, h = pl.pallas_call(
        _k1,
        grid=(NEB,),
        in_specs=[ebk, ebk, full((N, 128)), full((128, M)), full((1, M)),
                  full((M, M)), full((1, M)), full((2 * M, M)), full((1, M)),
                  full((2 * M, M)), full((1, M))],
        out_specs=[full((N, N)), full((N, N))] + [full((N, M))] * 6,
        out_shape=[jax.ShapeDtypeStruct((N, N), jnp.bfloat16)] * 2
        + [jax.ShapeDtypeStruct((N, M), f32)] * 6,
    )(row, col, feat, W0, b0.reshape(1, M), W1, b1.reshape(1, M),
      Wm1, bm1.reshape(1, M), Wm2, bm2.reshape(1, M))

    aval, xx = pl.pallas_call(
        _k2,
        grid=(NQB,),
        in_specs=[qbk, qbk, full((N, N)), full((N, N))]
        + [full((N, M))] * 6 + [full((M + 1, M)), full((1, M))],
        out_specs=[pl.BlockSpec((QBLK, M), lambda q: (q, 0))] * 2,
        out_shape=[jax.ShapeDtypeStruct((2 * P, M), f32)] * 2,
    )(iq, jq, adjf, adjt, u1, v1, u2, v2, w, h, Wm3, bm3.reshape(1, M))

    out = pl.pallas_call(
        _k3,
        in_specs=[full((2 * P, M)), full((2 * P, M)), full((2 * M, 1)),
                  full((1, 1))],
        out_specs=full((P, 1)),
        out_shape=jax.ShapeDtypeStruct((P, 1), f32),
    )(aval, xx, Wdir, bdir.reshape(1, 1))
    return out


# speedup vs baseline: 14.2620x; 1.0930x over previous
import jax
import jax.numpy as jnp
from jax.experimental import pallas as pl

N = 512
E = 8192
P = 4096
M = 20
EBLK = 2048
QBLK = 512
HQB = QBLK // 2
NEB = E // EBLK
NQB = (2 * P) // QBLK


def _k1(row_ref, col_ref, feat_ref, w0_ref, b0_ref, w1_ref, b1_ref,
        wm1_ref, bm1_ref, wm2_ref, bm2_ref,
        adjf_ref, adjt_ref, u1_ref, v1_ref, u2_ref, v2_ref, w_ref, h_ref):
    e = pl.program_id(0)
    rows = row_ref[0, 0, :]
    cols = col_ref[0, 0, :]
    ids = jax.lax.broadcasted_iota(jnp.int32, (EBLK, N), 1)
    ohr = (rows[:, None] == ids).astype(jnp.bfloat16)
    ohc = (cols[:, None] == ids).astype(jnp.bfloat16)
    dn = (((0,), (0,)), ((), ()))
    pa = jax.lax.dot_general(ohr, ohc, dn,
                             preferred_element_type=jnp.float32
                             ).astype(jnp.bfloat16)
    pt = jax.lax.dot_general(ohc, ohr, dn,
                             preferred_element_type=jnp.float32
                             ).astype(jnp.bfloat16)

    @pl.when(e == 0)
    def _():
        adjf_ref[...] = pa
        adjt_ref[...] = pt

    @pl.when(e > 0)
    def _():
        adjf_ref[...] += pa
        adjt_ref[...] += pt

    @pl.when(e == NEB - 1)
    def _():
        adjt = adjt_ref[...].astype(jnp.float32)
        deg = jnp.sum(adjt, axis=1) + 1.0
        dinv = 1.0 / jnp.sqrt(deg)
        ir = jax.lax.broadcasted_iota(jnp.int32, (N, N), 0)
        ic = jax.lax.broadcasted_iota(jnp.int32, (N, N), 1)
        eye = (ir == ic).astype(jnp.float32)
        prop = (adjt + eye) * (dinv[:, None] * dinv[None, :])
        f32 = jnp.float32
        h = jnp.dot(prop, jnp.dot(feat_ref[...], w0_ref[...],
                                  preferred_element_type=f32),
                    preferred_element_type=f32) + b0_ref[...]
        h = jnp.dot(prop, jnp.dot(h, w1_ref[...],
                                  preferred_element_type=f32),
                    preferred_element_type=f32) + b1_ref[...]
        wm1 = wm1_ref[...]
        wm2 = wm2_ref[...]
        u1 = jnp.dot(h, wm1[:M, :], preferred_element_type=f32) + bm1_ref[...]
        v1 = jnp.dot(h, wm1[M:, :], preferred_element_type=f32)
        u2 = jnp.dot(h, wm2[:M, :], preferred_element_type=f32)
        v2 = jnp.dot(h, wm2[M:, :], preferred_element_type=f32) + bm2_ref[...]
        u1_ref[...] = u1
        v1_ref[...] = v1
        u2_ref[...] = u2
        v2_ref[...] = v2
        w_ref[...] = v1 * u2
        h_ref[...] = h


def _k2(iq_ref, jq_ref, adjf_ref, adjt_ref, u1_ref, v1_ref, u2_ref, v2_ref,
        w_ref, h_ref, wm3_ref, bm3_ref, wdir_ref, bdir_ref, out_ref):
    f32 = jnp.float32
    iqv = iq_ref[0, 0, :]
    jqv = jq_ref[0, 0, :]
    ids = jax.lax.broadcasted_iota(jnp.int32, (QBLK, N), 1)
    ohi = (iqv[:, None] == ids)
    ohj = (jqv[:, None] == ids)
    ohif = ohi.astype(f32)
    ohjf = ohj.astype(f32)
    aqr = jnp.dot(ohi.astype(jnp.bfloat16), adjf_ref[...],
                  preferred_element_type=f32)
    aqc = jnp.dot(ohj.astype(jnp.bfloat16), adjt_ref[...],
                  preferred_element_type=f32)
    z = aqr * aqc
    cnt = jnp.sum(z, axis=1)
    s1 = jnp.dot(z, v1_ref[...], preferred_element_type=f32)
    s2 = jnp.dot(z, u2_ref[...], preferred_element_type=f32)
    sx = jnp.dot(z, w_ref[...], preferred_element_type=f32)
    u1q = jnp.dot(ohif, u1_ref[...], preferred_element_type=f32)
    v2q = jnp.dot(ohjf, v2_ref[...], preferred_element_type=f32)
    hr = jnp.dot(ohif, h_ref[...], preferred_element_type=f32)
    hc = jnp.dot(ohjf, h_ref[...], preferred_element_type=f32)
    adjflag = jnp.sum(aqr * ohjf, axis=1)
    pd = (u1q * v2q) * cnt[:, None] + u1q * s2 + v2q * s1 + sx
    union = ((cnt + adjflag) > 0.0).astype(f32)
    wm3 = wm3_ref[...]
    aval = ((jnp.dot(pd, wm3[:M, :], preferred_element_type=f32)
             + adjflag[:, None] * wm3[M, :][None, :] + bm3_ref[...])
            * union[:, None])
    xf = aval[0:HQB, :] * aval[HQB:QBLK, :]
    xx = hr[0:HQB, :] * hc[0:HQB, :]
    wdir = wdir_ref[...]
    out_ref[...] = (jnp.dot(xf, wdir[:M, :], preferred_element_type=f32)
                    + jnp.dot(xx, wdir[M:, :], preferred_element_type=f32)
                    + bdir_ref[...])


def kernel(x, ei, pos, feat, W0, b0, W1, b1, Wm1, bm1, Wm2, bm2,
           Wm3, bm3, Wdir, bdir):
    f32 = jnp.float32
    row = ei[0].reshape(NEB, 1, EBLK)
    col = ei[1].reshape(NEB, 1, EBLK)
    iq = jnp.concatenate([pos[:, 0].reshape(NQB, 1, HQB),
                          pos[:, 1].reshape(NQB, 1, HQB)], axis=2)
    jq = jnp.concatenate([pos[:, 1].reshape(NQB, 1, HQB),
                          pos[:, 0].reshape(NQB, 1, HQB)], axis=2)

    full = lambda shp: pl.BlockSpec(shp, lambda *_: tuple(0 for _ in shp))
    ebk = pl.BlockSpec((1, 1, EBLK), lambda e: (e, 0, 0))
    qbk = pl.BlockSpec((1, 1, QBLK), lambda q: (q, 0, 0))

    adjf, adjt, u1, v1, u2, v2, w, h = pl.pallas_call(
        _k1,
        grid=(NEB,),
        in_specs=[ebk, ebk, full((N, 128)), full((128, M)), full((1, M)),
                  full((M, M)), full((1, M)), full((2 * M, M)), full((1, M)),
                  full((2 * M, M)), full((1, M))],
        out_specs=[full((N, N)), full((N, N))] + [full((N, M))] * 6,
        out_shape=[jax.ShapeDtypeStruct((N, N), jnp.bfloat16)] * 2
        + [jax.ShapeDtypeStruct((N, M), f32)] * 6,
    )(row, col, feat, W0, b0.reshape(1, M), W1, b1.reshape(1, M),
      Wm1, bm1.reshape(1, M), Wm2, bm2.reshape(1, M))

    out = pl.pallas_call(
        _k2,
        grid=(NQB,),
        in_specs=[qbk, qbk, full((N, N)), full((N, N))]
        + [full((N, M))] * 6
        + [full((M + 1, M)), full((1, M)), full((2 * M, 1)), full((1, 1))],
        out_specs=pl.BlockSpec((HQB, 1), lambda q: (q, 0)),
        out_shape=jax.ShapeDtypeStruct((P, 1), f32),
    )(iq, jq, adjf, adjt, u1, v1, u2, v2, w, h, Wm3, bm3.reshape(1, M),
      Wdir, bdir.reshape(1, 1))
    return out
